# baseline (speedup 1.0000x reference)
import functools
import jax
import jax.numpy as jnp
from jax import lax
from jax.experimental import pallas as pl
from jax.experimental.pallas import tpu as pltpu
from jax.experimental.pallas import tpu_sc as plsc

_NC = 2
_NS = 16
_LANES = 16
_C = 4
_TC_BLK = 512
_SC_FRAC_BLKS = 4


def _tc_add(x_ref, w_ref, o_ref):
    o_ref[...] = x_ref[...] + w_ref[...]


def _sc_part(x, weight, seq_off, seq_rows):
    batch, seq_len, dim = x.shape
    nw = _NC * _NS
    rows = seq_rows // nw
    n_chunks = rows // _C
    cols = dim // _LANES

    mesh = plsc.VectorSubcoreMesh(core_axis_name="c", subcore_axis_name="s")

    @functools.partial(
        pl.kernel,
        mesh=mesh,
        out_type=jax.ShapeDtypeStruct((batch, seq_rows, dim), x.dtype),
        scratch_types=[
            pltpu.VMEM((2, batch, _C, dim), jnp.float32),
            pltpu.VMEM((2, batch, _C, dim), jnp.float32),
            pltpu.VMEM((2, _C, dim), jnp.float32),
            pltpu.SemaphoreType.DMA,
            pltpu.SemaphoreType.DMA,
            pltpu.SemaphoreType.DMA,
            pltpu.SemaphoreType.DMA,
        ],
    )
    def _sc_add(x_hbm, w_hbm, out_hbm, vx, vo, vw, sl0, sl1, ss0, ss1):
        wid = lax.axis_index("s") * _NC + lax.axis_index("c")
        s0 = wid * rows
        sls = (sl0, sl1)
        sss = (ss0, ss1)

        def issue_loads(g, p):
            s = s0 + g * _C
            for b in range(batch):
                pltpu.async_copy(
                    x_hbm.at[b, pl.ds(seq_off + s, _C), :], vx.at[p, b], sls[p]
                )
            pltpu.async_copy(w_hbm.at[pl.ds(seq_off + s, _C), :], vw.at[p], sls[p])

        def wait_loads(g, p):
            s = s0 + g * _C
            for b in range(batch):
                pltpu.make_async_copy(
                    x_hbm.at[b, pl.ds(seq_off + s, _C), :], vx.at[p, b], sls[p]
                ).wait()
            pltpu.make_async_copy(
                w_hbm.at[pl.ds(seq_off + s, _C), :], vw.at[p], sls[p]
            ).wait()

        def issue_stores(g, p):
            s = s0 + g * _C
            for b in range(batch):
                pltpu.async_copy(vo.at[p, b], out_hbm.at[b, pl.ds(s, _C), :], sss[p])

        def wait_stores(g, p):
            s = s0 + g * _C
            for b in range(batch):
                pltpu.make_async_copy(
                    vo.at[p, b], out_hbm.at[b, pl.ds(s, _C), :], sss[p]
                ).wait()

        issue_loads(0, 0)
        issue_loads(1, 1)

        def pair(i, carry):
            for p in range(2):
                g = 2 * i + p
                wait_loads(g, p)

                @pl.when(g >= 2)
                def _():
                    wait_stores(g - 2, p)

                def add_col(j, c2):
                    col = pl.ds(j * _LANES, _LANES)
                    for r in range(_C):
                        wv = vw[p, r, col]
                        for b in range(batch):
                            vo[p, b, r, col] = vx[p, b, r, col] + wv
                    return c2

                lax.fori_loop(0, cols, add_col, 0)
                issue_stores(g, p)

                @pl.when(g + 2 < n_chunks)
                def _():
                    issue_loads(g + 2, p)

            return carry

        lax.fori_loop(0, n_chunks // 2, pair, 0)
        wait_stores(n_chunks - 2, 0)
        wait_stores(n_chunks - 1, 1)

    return _sc_add(x, weight)


def kernel(x, weight):
    batch, seq_len, dim = x.shape
    n_blks = seq_len // _TC_BLK
    tc_blks = n_blks - _SC_FRAC_BLKS
    split = tc_blks * _TC_BLK
    w = weight[None, :seq_len, :]

    tc_out = pl.pallas_call(
        _tc_add,
        grid=(tc_blks,),
        in_specs=[
            pl.BlockSpec((batch, _TC_BLK, dim), lambda s: (0, s, 0)),
            pl.BlockSpec((None, _TC_BLK, dim), lambda s: (0, s, 0)),
        ],
        out_specs=pl.BlockSpec((batch, _TC_BLK, dim), lambda s: (0, s, 0)),
        out_shape=jax.ShapeDtypeStruct(x.shape, x.dtype),
    )(x, w)

    sc_out = _sc_part(x, weight[:seq_len], split, seq_len - split)
    return lax.dynamic_update_slice(tc_out, sc_out, (0, split, 0))


# speedup vs baseline: 1.0587x; 1.0587x over previous
import functools
import jax
import jax.numpy as jnp
from jax import lax
from jax.experimental import pallas as pl
from jax.experimental.pallas import tpu as pltpu
from jax.experimental.pallas import tpu_sc as plsc

_NC = 2
_NS = 16
_LANES = 16
_C = 4


def kernel(x, weight):
    batch, seq_len, dim = x.shape
    nw = _NC * _NS
    rows = seq_len // nw
    n_chunks = rows // _C
    cols = dim // _LANES

    mesh = plsc.VectorSubcoreMesh(core_axis_name="c", subcore_axis_name="s")

    @functools.partial(
        pl.kernel,
        mesh=mesh,
        out_type=jax.ShapeDtypeStruct(x.shape, x.dtype),
        scratch_types=[
            pltpu.VMEM((2, batch, _C, dim), jnp.float32),
            pltpu.VMEM((2, batch, _C, dim), jnp.float32),
            pltpu.VMEM((2, _C, dim), jnp.float32),
            pltpu.SemaphoreType.DMA,
            pltpu.SemaphoreType.DMA,
            pltpu.SemaphoreType.DMA,
            pltpu.SemaphoreType.DMA,
        ],
    )
    def _sc_add(x_hbm, w_hbm, out_hbm, vx, vo, vw, sl0, sl1, ss0, ss1):
        wid = lax.axis_index("s") * _NC + lax.axis_index("c")
        s0 = wid * rows
        sls = (sl0, sl1)
        sss = (ss0, ss1)

        def issue_loads(g, p):
            s = s0 + g * _C
            for b in range(batch):
                pltpu.async_copy(x_hbm.at[b, pl.ds(s, _C), :], vx.at[p, b], sls[p])
            pltpu.async_copy(w_hbm.at[pl.ds(s, _C), :], vw.at[p], sls[p])

        def wait_loads(g, p):
            s = s0 + g * _C
            for b in range(batch):
                pltpu.make_async_copy(
                    x_hbm.at[b, pl.ds(s, _C), :], vx.at[p, b], sls[p]
                ).wait()
            pltpu.make_async_copy(w_hbm.at[pl.ds(s, _C), :], vw.at[p], sls[p]).wait()

        def issue_stores(g, p):
            s = s0 + g * _C
            for b in range(batch):
                pltpu.async_copy(vo.at[p, b], out_hbm.at[b, pl.ds(s, _C), :], sss[p])

        def wait_stores(g, p):
            s = s0 + g * _C
            for b in range(batch):
                pltpu.make_async_copy(
                    vo.at[p, b], out_hbm.at[b, pl.ds(s, _C), :], sss[p]
                ).wait()

        issue_loads(0, 0)
        issue_loads(1, 1)

        def pair(i, carry):
            for p in range(2):
                g = 2 * i + p
                wait_loads(g, p)

                @pl.when(g >= 2)
                def _():
                    wait_stores(g - 2, p)

                def add_col(j, c2):
                    col = pl.ds(j * _LANES, _LANES)
                    for r in range(_C):
                        wv = vw[p, r, col]
                        for b in range(batch):
                            vo[p, b, r, col] = vx[p, b, r, col] + wv
                    return c2

                lax.fori_loop(0, cols, add_col, 0)
                issue_stores(g, p)

                @pl.when(g + 2 < n_chunks)
                def _():
                    issue_loads(g + 2, p)

            return carry

        lax.fori_loop(0, n_chunks // 2, pair, 0)
        wait_stores(n_chunks - 2, 0)
        wait_stores(n_chunks - 1, 1)

    return _sc_add(x, weight[:seq_len])


# speedup vs baseline: 1.4304x; 1.3511x over previous
import jax
import jax.numpy as jnp
from jax.experimental import pallas as pl


def _add_kernel(x_ref, w_ref, o_ref):
    o_ref[...] = x_ref[...] + w_ref[...]


def kernel(x, weight):
    batch, seq_len, dim = x.shape
    blk = 512
    grid = (seq_len // blk,)
    return pl.pallas_call(
        _add_kernel,
        grid=grid,
        in_specs=[
            pl.BlockSpec((batch, blk, dim), lambda s: (0, s, 0)),
            pl.BlockSpec((None, blk, dim), lambda s: (0, s, 0)),
        ],
        out_specs=pl.BlockSpec((batch, blk, dim), lambda s: (0, s, 0)),
        out_shape=jax.ShapeDtypeStruct(x.shape, x.dtype),
    )(x, weight[None, :seq_len, :])
